# baseline (speedup 1.0000x reference)
import functools

import jax
import jax.numpy as jnp
from jax import lax
from jax.experimental import pallas as pl
from jax.experimental.pallas import tpu as pltpu

_TOPK = 0.1
_TOPK_CHANNEL = 0.3
_LAMBDA_LOCALITY = 0.5
_LAMBDA_L1 = 1.0
_LAMBDA_COV = 1.0

_FBITS_HI = 0x7F800000


def _count_select(bits, kk, n_idx_iters, hw):
    r = bits.shape[0]

    def vbody(b, v):
        cand = v | (jnp.int32(1) << (30 - b))
        cnt = jnp.sum((bits >= cand).astype(jnp.int32), axis=1, keepdims=True)
        return jnp.where(cnt >= kk, cand, v)

    vstar = lax.fori_loop(0, 31, vbody, jnp.zeros((r, 1), jnp.int32))
    cnt_ge = jnp.sum((bits >= vstar).astype(jnp.int32), axis=1, keepdims=True)

    def tie_path():
        eq = bits == vstar
        cnt_gt = jnp.sum((bits > vstar).astype(jnp.int32), axis=1,
                         keepdims=True)
        t = kk - cnt_gt
        iota = lax.broadcasted_iota(jnp.int32, bits.shape, 1)
        lo2 = jnp.zeros((r, 1), jnp.int32)
        hi2 = jnp.full((r, 1), hw - 1, jnp.int32)

        def ibody(_, carry):
            lo_, hi_ = carry
            mid = lo_ + (hi_ - lo_) // 2
            cnt = jnp.sum((eq & (iota <= mid)).astype(jnp.int32), axis=1,
                          keepdims=True)
            pred = cnt >= t
            return jnp.where(pred, lo_, mid + 1), jnp.where(pred, mid, hi_)

        _, idx_cut = lax.fori_loop(0, n_idx_iters, ibody, (lo2, hi2))
        return idx_cut

    idx_cut = lax.cond(jnp.any(cnt_ge != kk), tie_path,
                       lambda: jnp.full((r, 1), hw - 1, jnp.int32))
    return vstar, idx_cut


def _phase_a(x_ref, vstar_ref, idxc_ref, chsum_ref, rowsum_ref, entx_ref,
             enty_ref, *, k, h, w, n_idx_iters):
    hw = h * w
    xb = x_ref[0].reshape(x_ref.shape[1], hw)
    xa = jnp.abs(xb)
    bits = lax.bitcast_convert_type(xa, jnp.int32)
    vstar, idx_cut = _count_select(bits, k, n_idx_iters, hw)

    iota = lax.broadcasted_iota(jnp.int32, bits.shape, 1)
    keep = (bits > vstar) | ((bits == vstar) & (iota <= idx_cut))
    keepf = keep.astype(jnp.float32)

    chsum = jnp.sum(keepf * xa, axis=1, keepdims=True)
    rowsum = jnp.sum(xa, axis=1, keepdims=True)

    ii = lax.broadcasted_iota(jnp.int32, (hw, h), 0)
    jh = lax.broadcasted_iota(jnp.int32, (hw, h), 1)
    a_ind = ((ii // w) == jh).astype(jnp.float32)
    ii2 = lax.broadcasted_iota(jnp.int32, (hw, w), 0)
    jw = lax.broadcasted_iota(jnp.int32, (hw, w), 1)
    b_ind = ((ii2 % w) == jw).astype(jnp.float32)
    xcp = jnp.dot(xa, a_ind, preferred_element_type=jnp.float32)
    ycp = jnp.dot(xa, b_ind, preferred_element_type=jnp.float32)
    logs = jnp.log(rowsum)
    entx = logs - jnp.sum(xcp * jnp.log(xcp), axis=1, keepdims=True) / rowsum
    enty = logs - jnp.sum(ycp * jnp.log(ycp), axis=1, keepdims=True) / rowsum

    vstar_ref[0] = vstar
    idxc_ref[0] = idx_cut
    chsum_ref[0] = chsum
    rowsum_ref[0] = rowsum
    entx_ref[0] = entx
    enty_ref[0] = enty


def _phase_b(chsum_ref, rowsum_ref, entx_ref, enty_ref, chmask_ref, wcoef_ref,
             regp_ref, *, k2, c, hw, n_idx_iters):
    chsum = chsum_ref[...]
    rowsum = rowsum_ref[...]
    total = jnp.sum(chsum, axis=1, keepdims=True)
    chprob = chsum / total
    pbits = lax.bitcast_convert_type(chprob, jnp.int32)
    vstar, idx_cut = _count_select(pbits, k2, n_idx_iters, c)
    iota = lax.broadcasted_iota(jnp.int32, pbits.shape, 1)
    keep = (pbits > vstar) | ((pbits == vstar) & (iota <= idx_cut))
    chmask = keep.astype(jnp.float32)
    chmask_ref[...] = chmask
    wcoef_ref[...] = chmask / rowsum

    samplesum = jnp.sum(rowsum, axis=1, keepdims=True)
    wmag = rowsum / samplesum
    nrows = chsum.shape[0] * c
    mx = jnp.sum(entx_ref[...] * wmag) / nrows
    my = jnp.sum(enty_ref[...] * wmag) / nrows
    l1 = jnp.sum(rowsum) / (nrows * hw)
    regp = l1 * _LAMBDA_L1 - (mx + my) * _LAMBDA_LOCALITY
    regp_ref[...] = regp.reshape(1, 1)


def _phase_c(x_ref, vstar_ref, idxc_ref, chm_ref, wco_ref, regp_ref,
             sparse_ref, reg_ref, tacc_ref, cov_ref, *, nn, ncb, hw):
    n = pl.program_id(0)
    cb = pl.program_id(1)
    cblk = x_ref.shape[1]
    xb = x_ref[0].reshape(cblk, hw)
    xa = jnp.abs(xb)
    bits = lax.bitcast_convert_type(xa, jnp.int32)
    vstar = vstar_ref[0]
    idx_cut = idxc_ref[0]
    chm = chm_ref[0]
    wco = wco_ref[0]
    iota = lax.broadcasted_iota(jnp.int32, bits.shape, 1)
    keep = (bits > vstar) | ((bits == vstar) & (iota <= idx_cut))
    sparse_ref[0] = (xb * keep.astype(jnp.float32) * chm).reshape(
        x_ref.shape[1:])

    part = jnp.sum(xa * wco, axis=0, keepdims=True)

    @pl.when(cb == 0)
    def _():
        tacc_ref[0:1, :] = part

    @pl.when(cb != 0)
    def _():
        tacc_ref[0:1, :] = tacc_ref[0:1, :] + part

    @pl.when(cb == ncb - 1)
    def _():
        tt = tacc_ref[0:1, :]
        covn = jnp.sum((tt - 1.0) ** 2).reshape(1, 1)

        @pl.when(n == 0)
        def _():
            cov_ref[0:1, 0:1] = covn

        @pl.when(n != 0)
        def _():
            cov_ref[0:1, 0:1] = cov_ref[0:1, 0:1] + covn

        @pl.when(n == nn - 1)
        def _():
            reg_ref[...] = (regp_ref[...]
                            + cov_ref[0:1, 0:1] * (_LAMBDA_COV / (nn * hw)))


def kernel(x, tau):
    del tau
    n, c, h, w = x.shape
    hw = h * w
    rows = n * c
    k = max(int(_TOPK * hw), 1)
    k2 = max(int(_TOPK_CHANNEL * c), 1)
    r_blk = 1024 if rows % 1024 == 0 else rows
    nb = rows // r_blk
    it_hw = max((hw - 1).bit_length(), 1)
    it_c = max((c - 1).bit_length(), 1)

    xflat = x.reshape(nb, r_blk, h, w)
    a_out = pl.pallas_call(
        functools.partial(_phase_a, k=k, h=h, w=w, n_idx_iters=it_hw),
        grid=(nb,),
        in_specs=[pl.BlockSpec((1, r_blk, h, w), lambda i: (i, 0, 0, 0))],
        out_specs=[pl.BlockSpec((1, r_blk, 1), lambda i: (i, 0, 0))] * 6,
        out_shape=[
            jax.ShapeDtypeStruct((nb, r_blk, 1), jnp.int32),
            jax.ShapeDtypeStruct((nb, r_blk, 1), jnp.int32),
            jax.ShapeDtypeStruct((nb, r_blk, 1), jnp.float32),
            jax.ShapeDtypeStruct((nb, r_blk, 1), jnp.float32),
            jax.ShapeDtypeStruct((nb, r_blk, 1), jnp.float32),
            jax.ShapeDtypeStruct((nb, r_blk, 1), jnp.float32),
        ],
    )(xflat)
    vstar, idx_cut, chsum, rowsum, entx, enty = a_out

    b_out = pl.pallas_call(
        functools.partial(_phase_b, k2=k2, c=c, hw=hw, n_idx_iters=it_c),
        out_shape=[
            jax.ShapeDtypeStruct((n, c), jnp.float32),
            jax.ShapeDtypeStruct((n, c), jnp.float32),
            jax.ShapeDtypeStruct((1, 1), jnp.float32),
        ],
    )(chsum.reshape(n, c), rowsum.reshape(n, c), entx.reshape(n, c),
      enty.reshape(n, c))
    chmask, wcoef, regpart = b_out

    cblk = 256 if c % 256 == 0 else c
    ncb = c // cblk
    nrb = rows // cblk
    sparse, reg = pl.pallas_call(
        functools.partial(_phase_c, nn=n, ncb=ncb, hw=hw),
        grid=(n, ncb),
        in_specs=[
            pl.BlockSpec((1, cblk, h, w), lambda i, j: (i, j, 0, 0)),
            pl.BlockSpec((1, cblk, 1), lambda i, j, _ncb=ncb: (i * _ncb + j, 0, 0)),
            pl.BlockSpec((1, cblk, 1), lambda i, j, _ncb=ncb: (i * _ncb + j, 0, 0)),
            pl.BlockSpec((1, cblk, 1), lambda i, j, _ncb=ncb: (i * _ncb + j, 0, 0)),
            pl.BlockSpec((1, cblk, 1), lambda i, j, _ncb=ncb: (i * _ncb + j, 0, 0)),
            pl.BlockSpec((1, 1), lambda i, j: (0, 0)),
        ],
        out_specs=[
            pl.BlockSpec((1, cblk, h, w), lambda i, j: (i, j, 0, 0)),
            pl.BlockSpec((1, 1), lambda i, j: (0, 0)),
        ],
        out_shape=[
            jax.ShapeDtypeStruct((n, c, h, w), jnp.float32),
            jax.ShapeDtypeStruct((1, 1), jnp.float32),
        ],
        scratch_shapes=[
            pltpu.VMEM((8, hw), jnp.float32),
            pltpu.VMEM((8, 128), jnp.float32),
        ],
    )(x, vstar.reshape(nrb, cblk, 1), idx_cut.reshape(nrb, cblk, 1),
      chmask.reshape(nrb, cblk, 1), wcoef.reshape(nrb, cblk, 1), regpart)

    return sparse, reg[0, 0]


# speedup vs baseline: 1.4379x; 1.4379x over previous
import functools

import jax
import jax.numpy as jnp
from jax import lax
from jax.experimental import pallas as pl
from jax.experimental.pallas import tpu as pltpu

_TOPK = 0.1
_TOPK_CHANNEL = 0.3
_LAMBDA_LOCALITY = 0.5
_LAMBDA_L1 = 1.0
_LAMBDA_COV = 1.0

_FBITS_HI = 0x7F800000


def _count_select(bits, kk, n_idx_iters, hw):
    r = bits.shape[0]

    def vbody(b, v):
        cand = v | (jnp.int32(1) << (30 - b))
        cnt = jnp.sum((bits >= cand).astype(jnp.int32), axis=1, keepdims=True)
        return jnp.where(cnt >= kk, cand, v)

    vstar = lax.fori_loop(0, 31, vbody, jnp.zeros((r, 1), jnp.int32))
    cnt_ge = jnp.sum((bits >= vstar).astype(jnp.int32), axis=1, keepdims=True)

    def tie_path():
        eq = bits == vstar
        cnt_gt = jnp.sum((bits > vstar).astype(jnp.int32), axis=1,
                         keepdims=True)
        t = kk - cnt_gt
        iota = lax.broadcasted_iota(jnp.int32, bits.shape, 1)
        lo2 = jnp.zeros((r, 1), jnp.int32)
        hi2 = jnp.full((r, 1), hw - 1, jnp.int32)

        def ibody(_, carry):
            lo_, hi_ = carry
            mid = lo_ + (hi_ - lo_) // 2
            cnt = jnp.sum((eq & (iota <= mid)).astype(jnp.int32), axis=1,
                          keepdims=True)
            pred = cnt >= t
            return jnp.where(pred, lo_, mid + 1), jnp.where(pred, mid, hi_)

        _, idx_cut = lax.fori_loop(0, n_idx_iters, ibody, (lo2, hi2))
        return idx_cut

    idx_cut = lax.cond(jnp.any(cnt_ge != kk), tie_path,
                       lambda: jnp.full((r, 1), hw - 1, jnp.int32))
    return vstar, idx_cut


def _phase_a(x_ref, vstar_ref, idxc_ref, chsum_ref, rowsum_ref, entx_ref,
             enty_ref, *, k, h, w, n_idx_iters):
    hw = h * w
    xb = x_ref[0]
    xa = jnp.abs(xb)
    bits = lax.bitcast_convert_type(xa, jnp.int32)
    vstar, idx_cut = _count_select(bits, k, n_idx_iters, hw)

    iota = lax.broadcasted_iota(jnp.int32, bits.shape, 1)
    keep = (bits > vstar) | ((bits == vstar) & (iota <= idx_cut))
    keepf = keep.astype(jnp.float32)

    chsum = jnp.sum(keepf * xa, axis=1, keepdims=True)
    rowsum = jnp.sum(xa, axis=1, keepdims=True)

    ii = lax.broadcasted_iota(jnp.int32, (hw, h), 0)
    jh = lax.broadcasted_iota(jnp.int32, (hw, h), 1)
    a_ind = ((ii // w) == jh).astype(jnp.float32)
    ii2 = lax.broadcasted_iota(jnp.int32, (hw, w), 0)
    jw = lax.broadcasted_iota(jnp.int32, (hw, w), 1)
    b_ind = ((ii2 % w) == jw).astype(jnp.float32)
    xcp = jnp.dot(xa, a_ind, preferred_element_type=jnp.float32)
    ycp = jnp.dot(xa, b_ind, preferred_element_type=jnp.float32)
    logs = jnp.log(rowsum)
    entx = logs - jnp.sum(xcp * jnp.log(xcp), axis=1, keepdims=True) / rowsum
    enty = logs - jnp.sum(ycp * jnp.log(ycp), axis=1, keepdims=True) / rowsum

    vstar_ref[0] = vstar
    idxc_ref[0] = idx_cut
    chsum_ref[0] = chsum
    rowsum_ref[0] = rowsum
    entx_ref[0] = entx
    enty_ref[0] = enty


def _phase_b(chsum_ref, rowsum_ref, entx_ref, enty_ref, chmask_ref, wcoef_ref,
             regp_ref, *, k2, c, hw, n_idx_iters):
    chsum = chsum_ref[...]
    rowsum = rowsum_ref[...]
    total = jnp.sum(chsum, axis=1, keepdims=True)
    chprob = chsum / total
    pbits = lax.bitcast_convert_type(chprob, jnp.int32)
    vstar, idx_cut = _count_select(pbits, k2, n_idx_iters, c)
    iota = lax.broadcasted_iota(jnp.int32, pbits.shape, 1)
    keep = (pbits > vstar) | ((pbits == vstar) & (iota <= idx_cut))
    chmask = keep.astype(jnp.float32)
    chmask_ref[...] = chmask
    wcoef_ref[...] = chmask / rowsum

    samplesum = jnp.sum(rowsum, axis=1, keepdims=True)
    wmag = rowsum / samplesum
    nrows = chsum.shape[0] * c
    mx = jnp.sum(entx_ref[...] * wmag) / nrows
    my = jnp.sum(enty_ref[...] * wmag) / nrows
    l1 = jnp.sum(rowsum) / (nrows * hw)
    regp = l1 * _LAMBDA_L1 - (mx + my) * _LAMBDA_LOCALITY
    regp_ref[...] = regp.reshape(1, 1)


def _phase_c(x_ref, vstar_ref, idxc_ref, chm_ref, wco_ref, regp_ref,
             sparse_ref, reg_ref, tacc_ref, cov_ref, *, nn, ncb, hw):
    n = pl.program_id(0)
    cb = pl.program_id(1)
    xb = x_ref[0]
    xa = jnp.abs(xb)
    bits = lax.bitcast_convert_type(xa, jnp.int32)
    vstar = vstar_ref[0]
    idx_cut = idxc_ref[0]
    chm = chm_ref[0]
    wco = wco_ref[0]
    iota = lax.broadcasted_iota(jnp.int32, bits.shape, 1)
    keep = (bits > vstar) | ((bits == vstar) & (iota <= idx_cut))
    sparse_ref[0] = xb * keep.astype(jnp.float32) * chm

    part = jnp.sum(xa * wco, axis=0, keepdims=True)

    @pl.when(cb == 0)
    def _():
        tacc_ref[0:1, :] = part

    @pl.when(cb != 0)
    def _():
        tacc_ref[0:1, :] = tacc_ref[0:1, :] + part

    @pl.when(cb == ncb - 1)
    def _():
        tt = tacc_ref[0:1, :]
        covn = jnp.sum((tt - 1.0) ** 2).reshape(1, 1)

        @pl.when(n == 0)
        def _():
            cov_ref[0:1, 0:1] = covn

        @pl.when(n != 0)
        def _():
            cov_ref[0:1, 0:1] = cov_ref[0:1, 0:1] + covn

        @pl.when(n == nn - 1)
        def _():
            reg_ref[...] = (regp_ref[...]
                            + cov_ref[0:1, 0:1] * (_LAMBDA_COV / (nn * hw)))


def kernel(x, tau):
    del tau
    n, c, h, w = x.shape
    hw = h * w
    rows = n * c
    k = max(int(_TOPK * hw), 1)
    k2 = max(int(_TOPK_CHANNEL * c), 1)
    r_blk = 1024 if rows % 1024 == 0 else rows
    nb = rows // r_blk
    it_hw = max((hw - 1).bit_length(), 1)
    it_c = max((c - 1).bit_length(), 1)

    xflat = x.reshape(nb, r_blk, hw)
    a_out = pl.pallas_call(
        functools.partial(_phase_a, k=k, h=h, w=w, n_idx_iters=it_hw),
        grid=(nb,),
        in_specs=[pl.BlockSpec((1, r_blk, hw), lambda i: (i, 0, 0))],
        out_specs=[pl.BlockSpec((1, r_blk, 1), lambda i: (i, 0, 0))] * 6,
        out_shape=[
            jax.ShapeDtypeStruct((nb, r_blk, 1), jnp.int32),
            jax.ShapeDtypeStruct((nb, r_blk, 1), jnp.int32),
            jax.ShapeDtypeStruct((nb, r_blk, 1), jnp.float32),
            jax.ShapeDtypeStruct((nb, r_blk, 1), jnp.float32),
            jax.ShapeDtypeStruct((nb, r_blk, 1), jnp.float32),
            jax.ShapeDtypeStruct((nb, r_blk, 1), jnp.float32),
        ],
    )(xflat)
    vstar, idx_cut, chsum, rowsum, entx, enty = a_out

    b_out = pl.pallas_call(
        functools.partial(_phase_b, k2=k2, c=c, hw=hw, n_idx_iters=it_c),
        out_shape=[
            jax.ShapeDtypeStruct((n, c), jnp.float32),
            jax.ShapeDtypeStruct((n, c), jnp.float32),
            jax.ShapeDtypeStruct((1, 1), jnp.float32),
        ],
    )(chsum.reshape(n, c), rowsum.reshape(n, c), entx.reshape(n, c),
      enty.reshape(n, c))
    chmask, wcoef, regpart = b_out

    cblk = 256 if c % 256 == 0 else c
    ncb = c // cblk
    nrb = rows // cblk
    x3 = x.reshape(n, c, hw)
    sparse, reg = pl.pallas_call(
        functools.partial(_phase_c, nn=n, ncb=ncb, hw=hw),
        grid=(n, ncb),
        in_specs=[
            pl.BlockSpec((1, cblk, hw), lambda i, j: (i, j, 0)),
            pl.BlockSpec((1, cblk, 1), lambda i, j, _ncb=ncb: (i * _ncb + j, 0, 0)),
            pl.BlockSpec((1, cblk, 1), lambda i, j, _ncb=ncb: (i * _ncb + j, 0, 0)),
            pl.BlockSpec((1, cblk, 1), lambda i, j, _ncb=ncb: (i * _ncb + j, 0, 0)),
            pl.BlockSpec((1, cblk, 1), lambda i, j, _ncb=ncb: (i * _ncb + j, 0, 0)),
            pl.BlockSpec((1, 1), lambda i, j: (0, 0)),
        ],
        out_specs=[
            pl.BlockSpec((1, cblk, hw), lambda i, j: (i, j, 0)),
            pl.BlockSpec((1, 1), lambda i, j: (0, 0)),
        ],
        out_shape=[
            jax.ShapeDtypeStruct((n, c, hw), jnp.float32),
            jax.ShapeDtypeStruct((1, 1), jnp.float32),
        ],
        scratch_shapes=[
            pltpu.VMEM((8, hw), jnp.float32),
            pltpu.VMEM((8, 128), jnp.float32),
        ],
    )(x3, vstar.reshape(nrb, cblk, 1), idx_cut.reshape(nrb, cblk, 1),
      chmask.reshape(nrb, cblk, 1), wcoef.reshape(nrb, cblk, 1), regpart)

    return sparse.reshape(n, c, h, w), reg[0, 0]


# speedup vs baseline: 1.5313x; 1.0650x over previous
import functools

import jax
import jax.numpy as jnp
from jax import lax
from jax.experimental import pallas as pl
from jax.experimental.pallas import tpu as pltpu

_TOPK = 0.1
_TOPK_CHANNEL = 0.3
_LAMBDA_LOCALITY = 0.5
_LAMBDA_L1 = 1.0
_LAMBDA_COV = 1.0

_FBITS_HI = 0x7F800000


def _count_select(bits, kk, n_idx_iters, hw):
    r = bits.shape[0]

    def vbody(b, v):
        cand = v | (jnp.int32(1) << (30 - b))
        cnt = jnp.sum((bits >= cand).astype(jnp.int32), axis=1, keepdims=True)
        return jnp.where(cnt >= kk, cand, v)

    vstar = lax.fori_loop(0, 31, vbody, jnp.zeros((r, 1), jnp.int32))
    cnt_ge = jnp.sum((bits >= vstar).astype(jnp.int32), axis=1, keepdims=True)

    def tie_path():
        eq = bits == vstar
        cnt_gt = jnp.sum((bits > vstar).astype(jnp.int32), axis=1,
                         keepdims=True)
        t = kk - cnt_gt
        iota = lax.broadcasted_iota(jnp.int32, bits.shape, 1)
        lo2 = jnp.zeros((r, 1), jnp.int32)
        hi2 = jnp.full((r, 1), hw - 1, jnp.int32)

        def ibody(_, carry):
            lo_, hi_ = carry
            mid = lo_ + (hi_ - lo_) // 2
            cnt = jnp.sum((eq & (iota <= mid)).astype(jnp.int32), axis=1,
                          keepdims=True)
            pred = cnt >= t
            return jnp.where(pred, lo_, mid + 1), jnp.where(pred, mid, hi_)

        _, idx_cut = lax.fori_loop(0, n_idx_iters, ibody, (lo2, hi2))
        return idx_cut

    idx_cut = lax.cond(jnp.any(cnt_ge != kk), tie_path,
                       lambda: jnp.full((r, 1), hw - 1, jnp.int32))
    return vstar, idx_cut


def _count_select_i16(bits, kk, n_idx_iters, hw):
    r = bits.shape[0]

    def rowcount(m16):
        s = m16
        while s.shape[1] > 128:
            half = s.shape[1] // 2
            s = s[:, :half] + s[:, half:]
        return jnp.sum(s.astype(jnp.int32), axis=1, keepdims=True)

    hi = ((bits >> 15) - 32768).astype(jnp.int16)

    def vbody1(b, v):
        cand = v | (jnp.int32(1) << (15 - b))
        cand16 = (cand - 32768).astype(jnp.int16)
        cnt = rowcount((hi >= cand16).astype(jnp.int16))
        return jnp.where(cnt >= kk, cand, v)

    vhi = lax.fori_loop(0, 16, vbody1, jnp.zeros((r, 1), jnp.int32))
    vhi16 = (vhi - 32768).astype(jnp.int16)

    lo16 = (bits & 0x7FFF).astype(jnp.int16)
    y = jnp.where(hi > vhi16, jnp.int16(0x7FFF),
                  jnp.where(hi == vhi16, lo16, jnp.int16(-1)))

    def vbody2(b, v):
        cand = v | (jnp.int32(1) << (14 - b))
        cand16 = cand.astype(jnp.int16)
        cnt = rowcount((y >= cand16).astype(jnp.int16))
        return jnp.where(cnt >= kk, cand, v)

    vlo = lax.fori_loop(0, 15, vbody2, jnp.zeros((r, 1), jnp.int32))
    vstar = (vhi << 15) | vlo
    cnt_ge = rowcount((y >= vlo.astype(jnp.int16)).astype(jnp.int16))

    def tie_path():
        eq = bits == vstar
        cnt_gt = jnp.sum((bits > vstar).astype(jnp.int32), axis=1,
                         keepdims=True)
        t = kk - cnt_gt
        iota = lax.broadcasted_iota(jnp.int32, bits.shape, 1)
        lo2 = jnp.zeros((r, 1), jnp.int32)
        hi2 = jnp.full((r, 1), hw - 1, jnp.int32)

        def ibody(_, carry):
            lo_, hi_ = carry
            mid = lo_ + (hi_ - lo_) // 2
            cnt = jnp.sum((eq & (iota <= mid)).astype(jnp.int32), axis=1,
                          keepdims=True)
            pred = cnt >= t
            return jnp.where(pred, lo_, mid + 1), jnp.where(pred, mid, hi_)

        _, idx_cut = lax.fori_loop(0, n_idx_iters, ibody, (lo2, hi2))
        return idx_cut

    idx_cut = lax.cond(jnp.any(cnt_ge != kk), tie_path,
                       lambda: jnp.full((r, 1), hw - 1, jnp.int32))
    return vstar, idx_cut


def _phase_a(x_ref, vstar_ref, idxc_ref, chsum_ref, rowsum_ref, entx_ref,
             enty_ref, *, k, h, w, n_idx_iters):
    hw = h * w
    xb = x_ref[0]
    xa = jnp.abs(xb)
    bits = lax.bitcast_convert_type(xa, jnp.int32)
    vstar, idx_cut = _count_select_i16(bits, k, n_idx_iters, hw)

    iota = lax.broadcasted_iota(jnp.int32, bits.shape, 1)
    keep = (bits > vstar) | ((bits == vstar) & (iota <= idx_cut))
    keepf = keep.astype(jnp.float32)

    chsum = jnp.sum(keepf * xa, axis=1, keepdims=True)
    rowsum = jnp.sum(xa, axis=1, keepdims=True)

    ii = lax.broadcasted_iota(jnp.int32, (hw, h), 0)
    jh = lax.broadcasted_iota(jnp.int32, (hw, h), 1)
    a_ind = ((ii // w) == jh).astype(jnp.float32)
    ii2 = lax.broadcasted_iota(jnp.int32, (hw, w), 0)
    jw = lax.broadcasted_iota(jnp.int32, (hw, w), 1)
    b_ind = ((ii2 % w) == jw).astype(jnp.float32)
    xcp = jnp.dot(xa, a_ind, preferred_element_type=jnp.float32)
    ycp = jnp.dot(xa, b_ind, preferred_element_type=jnp.float32)
    logs = jnp.log(rowsum)
    entx = logs - jnp.sum(xcp * jnp.log(xcp), axis=1, keepdims=True) / rowsum
    enty = logs - jnp.sum(ycp * jnp.log(ycp), axis=1, keepdims=True) / rowsum

    vstar_ref[0] = vstar
    idxc_ref[0] = idx_cut
    chsum_ref[0] = chsum
    rowsum_ref[0] = rowsum
    entx_ref[0] = entx
    enty_ref[0] = enty


def _phase_b(chsum_ref, rowsum_ref, entx_ref, enty_ref, chmask_ref, wcoef_ref,
             regp_ref, *, k2, c, hw, n_idx_iters):
    chsum = chsum_ref[...]
    rowsum = rowsum_ref[...]
    total = jnp.sum(chsum, axis=1, keepdims=True)
    chprob = chsum / total
    pbits = lax.bitcast_convert_type(chprob, jnp.int32)
    vstar, idx_cut = _count_select(pbits, k2, n_idx_iters, c)
    iota = lax.broadcasted_iota(jnp.int32, pbits.shape, 1)
    keep = (pbits > vstar) | ((pbits == vstar) & (iota <= idx_cut))
    chmask = keep.astype(jnp.float32)
    chmask_ref[...] = chmask
    wcoef_ref[...] = chmask / rowsum

    samplesum = jnp.sum(rowsum, axis=1, keepdims=True)
    wmag = rowsum / samplesum
    nrows = chsum.shape[0] * c
    mx = jnp.sum(entx_ref[...] * wmag) / nrows
    my = jnp.sum(enty_ref[...] * wmag) / nrows
    l1 = jnp.sum(rowsum) / (nrows * hw)
    regp = l1 * _LAMBDA_L1 - (mx + my) * _LAMBDA_LOCALITY
    regp_ref[...] = regp.reshape(1, 1)


def _phase_c(x_ref, vstar_ref, idxc_ref, chm_ref, wco_ref, regp_ref,
             sparse_ref, reg_ref, tacc_ref, cov_ref, *, nn, ncb, hw):
    n = pl.program_id(0)
    cb = pl.program_id(1)
    xb = x_ref[0]
    xa = jnp.abs(xb)
    bits = lax.bitcast_convert_type(xa, jnp.int32)
    vstar = vstar_ref[0]
    idx_cut = idxc_ref[0]
    chm = chm_ref[0]
    wco = wco_ref[0]
    iota = lax.broadcasted_iota(jnp.int32, bits.shape, 1)
    keep = (bits > vstar) | ((bits == vstar) & (iota <= idx_cut))
    sparse_ref[0] = xb * keep.astype(jnp.float32) * chm

    part = jnp.sum(xa * wco, axis=0, keepdims=True)

    @pl.when(cb == 0)
    def _():
        tacc_ref[0:1, :] = part

    @pl.when(cb != 0)
    def _():
        tacc_ref[0:1, :] = tacc_ref[0:1, :] + part

    @pl.when(cb == ncb - 1)
    def _():
        tt = tacc_ref[0:1, :]
        covn = jnp.sum((tt - 1.0) ** 2).reshape(1, 1)

        @pl.when(n == 0)
        def _():
            cov_ref[0:1, 0:1] = covn

        @pl.when(n != 0)
        def _():
            cov_ref[0:1, 0:1] = cov_ref[0:1, 0:1] + covn

        @pl.when(n == nn - 1)
        def _():
            reg_ref[...] = (regp_ref[...]
                            + cov_ref[0:1, 0:1] * (_LAMBDA_COV / (nn * hw)))


def kernel(x, tau):
    del tau
    n, c, h, w = x.shape
    hw = h * w
    rows = n * c
    k = max(int(_TOPK * hw), 1)
    k2 = max(int(_TOPK_CHANNEL * c), 1)
    r_blk = 1024 if rows % 1024 == 0 else rows
    nb = rows // r_blk
    it_hw = max((hw - 1).bit_length(), 1)
    it_c = max((c - 1).bit_length(), 1)

    xflat = x.reshape(nb, r_blk, hw)
    a_out = pl.pallas_call(
        functools.partial(_phase_a, k=k, h=h, w=w, n_idx_iters=it_hw),
        grid=(nb,),
        in_specs=[pl.BlockSpec((1, r_blk, hw), lambda i: (i, 0, 0))],
        out_specs=[pl.BlockSpec((1, r_blk, 1), lambda i: (i, 0, 0))] * 6,
        out_shape=[
            jax.ShapeDtypeStruct((nb, r_blk, 1), jnp.int32),
            jax.ShapeDtypeStruct((nb, r_blk, 1), jnp.int32),
            jax.ShapeDtypeStruct((nb, r_blk, 1), jnp.float32),
            jax.ShapeDtypeStruct((nb, r_blk, 1), jnp.float32),
            jax.ShapeDtypeStruct((nb, r_blk, 1), jnp.float32),
            jax.ShapeDtypeStruct((nb, r_blk, 1), jnp.float32),
        ],
    )(xflat)
    vstar, idx_cut, chsum, rowsum, entx, enty = a_out

    b_out = pl.pallas_call(
        functools.partial(_phase_b, k2=k2, c=c, hw=hw, n_idx_iters=it_c),
        out_shape=[
            jax.ShapeDtypeStruct((n, c), jnp.float32),
            jax.ShapeDtypeStruct((n, c), jnp.float32),
            jax.ShapeDtypeStruct((1, 1), jnp.float32),
        ],
    )(chsum.reshape(n, c), rowsum.reshape(n, c), entx.reshape(n, c),
      enty.reshape(n, c))
    chmask, wcoef, regpart = b_out

    cblk = 256 if c % 256 == 0 else c
    ncb = c // cblk
    nrb = rows // cblk
    x3 = x.reshape(n, c, hw)
    sparse, reg = pl.pallas_call(
        functools.partial(_phase_c, nn=n, ncb=ncb, hw=hw),
        grid=(n, ncb),
        in_specs=[
            pl.BlockSpec((1, cblk, hw), lambda i, j: (i, j, 0)),
            pl.BlockSpec((1, cblk, 1), lambda i, j, _ncb=ncb: (i * _ncb + j, 0, 0)),
            pl.BlockSpec((1, cblk, 1), lambda i, j, _ncb=ncb: (i * _ncb + j, 0, 0)),
            pl.BlockSpec((1, cblk, 1), lambda i, j, _ncb=ncb: (i * _ncb + j, 0, 0)),
            pl.BlockSpec((1, cblk, 1), lambda i, j, _ncb=ncb: (i * _ncb + j, 0, 0)),
            pl.BlockSpec((1, 1), lambda i, j: (0, 0)),
        ],
        out_specs=[
            pl.BlockSpec((1, cblk, hw), lambda i, j: (i, j, 0)),
            pl.BlockSpec((1, 1), lambda i, j: (0, 0)),
        ],
        out_shape=[
            jax.ShapeDtypeStruct((n, c, hw), jnp.float32),
            jax.ShapeDtypeStruct((1, 1), jnp.float32),
        ],
        scratch_shapes=[
            pltpu.VMEM((8, hw), jnp.float32),
            pltpu.VMEM((8, 128), jnp.float32),
        ],
    )(x3, vstar.reshape(nrb, cblk, 1), idx_cut.reshape(nrb, cblk, 1),
      chmask.reshape(nrb, cblk, 1), wcoef.reshape(nrb, cblk, 1), regpart)

    return sparse.reshape(n, c, h, w), reg[0, 0]


# speedup vs baseline: 1.6219x; 1.0591x over previous
import functools

import jax
import jax.numpy as jnp
from jax import lax
from jax.experimental import pallas as pl
from jax.experimental.pallas import tpu as pltpu

_TOPK = 0.1
_TOPK_CHANNEL = 0.3
_LAMBDA_LOCALITY = 0.5
_LAMBDA_L1 = 1.0
_LAMBDA_COV = 1.0

_FBITS_HI = 0x7F800000


def _count_select(bits, kk, n_idx_iters, hw):
    r = bits.shape[0]

    def vbody(b, v):
        cand = v | (jnp.int32(1) << (30 - b))
        cnt = jnp.sum((bits >= cand).astype(jnp.int32), axis=1, keepdims=True)
        return jnp.where(cnt >= kk, cand, v)

    vstar = lax.fori_loop(0, 31, vbody, jnp.zeros((r, 1), jnp.int32))
    cnt_ge = jnp.sum((bits >= vstar).astype(jnp.int32), axis=1, keepdims=True)

    def tie_path():
        eq = bits == vstar
        cnt_gt = jnp.sum((bits > vstar).astype(jnp.int32), axis=1,
                         keepdims=True)
        t = kk - cnt_gt
        iota = lax.broadcasted_iota(jnp.int32, bits.shape, 1)
        lo2 = jnp.zeros((r, 1), jnp.int32)
        hi2 = jnp.full((r, 1), hw - 1, jnp.int32)

        def ibody(_, carry):
            lo_, hi_ = carry
            mid = lo_ + (hi_ - lo_) // 2
            cnt = jnp.sum((eq & (iota <= mid)).astype(jnp.int32), axis=1,
                          keepdims=True)
            pred = cnt >= t
            return jnp.where(pred, lo_, mid + 1), jnp.where(pred, mid, hi_)

        _, idx_cut = lax.fori_loop(0, n_idx_iters, ibody, (lo2, hi2))
        return idx_cut

    idx_cut = lax.cond(jnp.any(cnt_ge != kk), tie_path,
                       lambda: jnp.full((r, 1), hw - 1, jnp.int32))
    return vstar, idx_cut


def _count_select_i16(bits, kk, n_idx_iters, hw):
    r = bits.shape[0]

    ones128 = jnp.ones((128, 1), jnp.float32)

    def rowcount(m16):
        s = m16
        while s.shape[1] > 128:
            half = s.shape[1] // 2
            s = s[:, :half] + s[:, half:]
        return jnp.dot(s.astype(jnp.float32), ones128,
                       preferred_element_type=jnp.float32)

    hi = ((bits >> 15) - 32768).astype(jnp.int16)

    def vbody1(b, v):
        cand = v | (jnp.int32(1) << (15 - b))
        cand16 = (cand - 32768).astype(jnp.int16)
        cnt = rowcount((hi >= cand16).astype(jnp.int16))
        return jnp.where(cnt >= kk, cand, v)

    vhi = lax.fori_loop(0, 16, vbody1, jnp.zeros((r, 1), jnp.int32))
    vhi16 = (vhi - 32768).astype(jnp.int16)

    lo16 = (bits & 0x7FFF).astype(jnp.int16)
    y = jnp.where(hi > vhi16, jnp.int16(0x7FFF),
                  jnp.where(hi == vhi16, lo16, jnp.int16(-1)))

    def vbody2(b, v):
        cand = v | (jnp.int32(1) << (14 - b))
        cand16 = cand.astype(jnp.int16)
        cnt = rowcount((y >= cand16).astype(jnp.int16))
        return jnp.where(cnt >= kk, cand, v)

    vlo = lax.fori_loop(0, 15, vbody2, jnp.zeros((r, 1), jnp.int32))
    vstar = (vhi << 15) | vlo
    cnt_ge = rowcount((y >= vlo.astype(jnp.int16)).astype(jnp.int16))

    def tie_path():
        eq = bits == vstar
        cnt_gt = jnp.sum((bits > vstar).astype(jnp.int32), axis=1,
                         keepdims=True)
        t = kk - cnt_gt
        iota = lax.broadcasted_iota(jnp.int32, bits.shape, 1)
        lo2 = jnp.zeros((r, 1), jnp.int32)
        hi2 = jnp.full((r, 1), hw - 1, jnp.int32)

        def ibody(_, carry):
            lo_, hi_ = carry
            mid = lo_ + (hi_ - lo_) // 2
            cnt = jnp.sum((eq & (iota <= mid)).astype(jnp.int32), axis=1,
                          keepdims=True)
            pred = cnt >= t
            return jnp.where(pred, lo_, mid + 1), jnp.where(pred, mid, hi_)

        _, idx_cut = lax.fori_loop(0, n_idx_iters, ibody, (lo2, hi2))
        return idx_cut

    idx_cut = lax.cond(jnp.any(cnt_ge != kk), tie_path,
                       lambda: jnp.full((r, 1), hw - 1, jnp.int32))
    return vstar, idx_cut


def _phase_a(x_ref, vstar_ref, idxc_ref, chsum_ref, rowsum_ref, entx_ref,
             enty_ref, *, k, h, w, n_idx_iters):
    hw = h * w
    xb = x_ref[0]
    xa = jnp.abs(xb)
    bits = lax.bitcast_convert_type(xa, jnp.int32)
    vstar, idx_cut = _count_select_i16(bits, k, n_idx_iters, hw)

    iota = lax.broadcasted_iota(jnp.int32, bits.shape, 1)
    keep = (bits > vstar) | ((bits == vstar) & (iota <= idx_cut))

    oneshw = jnp.ones((hw, 1), jnp.float32)
    chsum = jnp.dot(jnp.where(keep, xa, 0.0), oneshw,
                    preferred_element_type=jnp.float32)
    rowsum = jnp.dot(xa, oneshw, preferred_element_type=jnp.float32)

    ii = lax.broadcasted_iota(jnp.int32, (hw, h), 0)
    jh = lax.broadcasted_iota(jnp.int32, (hw, h), 1)
    a_ind = ((ii // w) == jh).astype(jnp.float32)
    ii2 = lax.broadcasted_iota(jnp.int32, (hw, w), 0)
    jw = lax.broadcasted_iota(jnp.int32, (hw, w), 1)
    b_ind = ((ii2 % w) == jw).astype(jnp.float32)
    xcp = jnp.dot(xa, a_ind, preferred_element_type=jnp.float32)
    ycp = jnp.dot(xa, b_ind, preferred_element_type=jnp.float32)
    logs = jnp.log(rowsum)
    entx = logs - jnp.sum(xcp * jnp.log(xcp), axis=1, keepdims=True) / rowsum
    enty = logs - jnp.sum(ycp * jnp.log(ycp), axis=1, keepdims=True) / rowsum

    vstar_ref[0] = vstar
    idxc_ref[0] = idx_cut
    chsum_ref[0] = chsum
    rowsum_ref[0] = rowsum
    entx_ref[0] = entx
    enty_ref[0] = enty


def _phase_b(chsum_ref, rowsum_ref, entx_ref, enty_ref, chmask_ref, wcoef_ref,
             regp_ref, *, k2, c, hw, n_idx_iters):
    chsum = chsum_ref[...]
    rowsum = rowsum_ref[...]
    total = jnp.sum(chsum, axis=1, keepdims=True)
    chprob = chsum / total
    pbits = lax.bitcast_convert_type(chprob, jnp.int32)
    vstar, idx_cut = _count_select(pbits, k2, n_idx_iters, c)
    iota = lax.broadcasted_iota(jnp.int32, pbits.shape, 1)
    keep = (pbits > vstar) | ((pbits == vstar) & (iota <= idx_cut))
    chmask = keep.astype(jnp.float32)
    chmask_ref[...] = chmask
    wcoef_ref[...] = chmask / rowsum

    samplesum = jnp.sum(rowsum, axis=1, keepdims=True)
    wmag = rowsum / samplesum
    nrows = chsum.shape[0] * c
    mx = jnp.sum(entx_ref[...] * wmag) / nrows
    my = jnp.sum(enty_ref[...] * wmag) / nrows
    l1 = jnp.sum(rowsum) / (nrows * hw)
    regp = l1 * _LAMBDA_L1 - (mx + my) * _LAMBDA_LOCALITY
    regp_ref[...] = regp.reshape(1, 1)


def _phase_c(x_ref, vstar_ref, idxc_ref, chm_ref, wco_ref, regp_ref,
             sparse_ref, reg_ref, tacc_ref, cov_ref, *, nn, ncb, hw):
    n = pl.program_id(0)
    cb = pl.program_id(1)
    xb = x_ref[0]
    xa = jnp.abs(xb)
    bits = lax.bitcast_convert_type(xa, jnp.int32)
    vstar = vstar_ref[0]
    idx_cut = idxc_ref[0]
    chm = chm_ref[0]
    wco = wco_ref[0]
    iota = lax.broadcasted_iota(jnp.int32, bits.shape, 1)
    keep = (bits > vstar) | ((bits == vstar) & (iota <= idx_cut))
    sparse_ref[0] = xb * keep.astype(jnp.float32) * chm

    part = jnp.sum(xa * wco, axis=0, keepdims=True)

    @pl.when(cb == 0)
    def _():
        tacc_ref[0:1, :] = part

    @pl.when(cb != 0)
    def _():
        tacc_ref[0:1, :] = tacc_ref[0:1, :] + part

    @pl.when(cb == ncb - 1)
    def _():
        tt = tacc_ref[0:1, :]
        covn = jnp.sum((tt - 1.0) ** 2).reshape(1, 1)

        @pl.when(n == 0)
        def _():
            cov_ref[0:1, 0:1] = covn

        @pl.when(n != 0)
        def _():
            cov_ref[0:1, 0:1] = cov_ref[0:1, 0:1] + covn

        @pl.when(n == nn - 1)
        def _():
            reg_ref[...] = (regp_ref[...]
                            + cov_ref[0:1, 0:1] * (_LAMBDA_COV / (nn * hw)))


def kernel(x, tau):
    del tau
    n, c, h, w = x.shape
    hw = h * w
    rows = n * c
    k = max(int(_TOPK * hw), 1)
    k2 = max(int(_TOPK_CHANNEL * c), 1)
    r_blk = 1024 if rows % 1024 == 0 else rows
    nb = rows // r_blk
    it_hw = max((hw - 1).bit_length(), 1)
    it_c = max((c - 1).bit_length(), 1)

    xflat = x.reshape(nb, r_blk, hw)
    a_out = pl.pallas_call(
        functools.partial(_phase_a, k=k, h=h, w=w, n_idx_iters=it_hw),
        grid=(nb,),
        in_specs=[pl.BlockSpec((1, r_blk, hw), lambda i: (i, 0, 0))],
        out_specs=[pl.BlockSpec((1, r_blk, 1), lambda i: (i, 0, 0))] * 6,
        out_shape=[
            jax.ShapeDtypeStruct((nb, r_blk, 1), jnp.int32),
            jax.ShapeDtypeStruct((nb, r_blk, 1), jnp.int32),
            jax.ShapeDtypeStruct((nb, r_blk, 1), jnp.float32),
            jax.ShapeDtypeStruct((nb, r_blk, 1), jnp.float32),
            jax.ShapeDtypeStruct((nb, r_blk, 1), jnp.float32),
            jax.ShapeDtypeStruct((nb, r_blk, 1), jnp.float32),
        ],
    )(xflat)
    vstar, idx_cut, chsum, rowsum, entx, enty = a_out

    b_out = pl.pallas_call(
        functools.partial(_phase_b, k2=k2, c=c, hw=hw, n_idx_iters=it_c),
        out_shape=[
            jax.ShapeDtypeStruct((n, c), jnp.float32),
            jax.ShapeDtypeStruct((n, c), jnp.float32),
            jax.ShapeDtypeStruct((1, 1), jnp.float32),
        ],
    )(chsum.reshape(n, c), rowsum.reshape(n, c), entx.reshape(n, c),
      enty.reshape(n, c))
    chmask, wcoef, regpart = b_out

    cblk = 256 if c % 256 == 0 else c
    ncb = c // cblk
    nrb = rows // cblk
    x3 = x.reshape(n, c, hw)
    sparse, reg = pl.pallas_call(
        functools.partial(_phase_c, nn=n, ncb=ncb, hw=hw),
        grid=(n, ncb),
        in_specs=[
            pl.BlockSpec((1, cblk, hw), lambda i, j: (i, j, 0)),
            pl.BlockSpec((1, cblk, 1), lambda i, j, _ncb=ncb: (i * _ncb + j, 0, 0)),
            pl.BlockSpec((1, cblk, 1), lambda i, j, _ncb=ncb: (i * _ncb + j, 0, 0)),
            pl.BlockSpec((1, cblk, 1), lambda i, j, _ncb=ncb: (i * _ncb + j, 0, 0)),
            pl.BlockSpec((1, cblk, 1), lambda i, j, _ncb=ncb: (i * _ncb + j, 0, 0)),
            pl.BlockSpec((1, 1), lambda i, j: (0, 0)),
        ],
        out_specs=[
            pl.BlockSpec((1, cblk, hw), lambda i, j: (i, j, 0)),
            pl.BlockSpec((1, 1), lambda i, j: (0, 0)),
        ],
        out_shape=[
            jax.ShapeDtypeStruct((n, c, hw), jnp.float32),
            jax.ShapeDtypeStruct((1, 1), jnp.float32),
        ],
        scratch_shapes=[
            pltpu.VMEM((8, hw), jnp.float32),
            pltpu.VMEM((8, 128), jnp.float32),
        ],
    )(x3, vstar.reshape(nrb, cblk, 1), idx_cut.reshape(nrb, cblk, 1),
      chmask.reshape(nrb, cblk, 1), wcoef.reshape(nrb, cblk, 1), regpart)

    return sparse.reshape(n, c, h, w), reg[0, 0]


# speedup vs baseline: 1.6345x; 1.0078x over previous
import functools

import jax
import jax.numpy as jnp
from jax import lax
from jax.experimental import pallas as pl
from jax.experimental.pallas import tpu as pltpu

_TOPK = 0.1
_TOPK_CHANNEL = 0.3
_LAMBDA_LOCALITY = 0.5
_LAMBDA_L1 = 1.0
_LAMBDA_COV = 1.0

_FBITS_HI = 0x7F800000


def _count_select(bits, kk, n_idx_iters, hw):
    r = bits.shape[0]

    def vbody(b, v):
        cand = v | (jnp.int32(1) << (30 - b))
        cnt = jnp.sum((bits >= cand).astype(jnp.int32), axis=1, keepdims=True)
        return jnp.where(cnt >= kk, cand, v)

    vstar = lax.fori_loop(0, 31, vbody, jnp.zeros((r, 1), jnp.int32))
    cnt_ge = jnp.sum((bits >= vstar).astype(jnp.int32), axis=1, keepdims=True)

    def tie_path():
        eq = bits == vstar
        cnt_gt = jnp.sum((bits > vstar).astype(jnp.int32), axis=1,
                         keepdims=True)
        t = kk - cnt_gt
        iota = lax.broadcasted_iota(jnp.int32, bits.shape, 1)
        lo2 = jnp.zeros((r, 1), jnp.int32)
        hi2 = jnp.full((r, 1), hw - 1, jnp.int32)

        def ibody(_, carry):
            lo_, hi_ = carry
            mid = lo_ + (hi_ - lo_) // 2
            cnt = jnp.sum((eq & (iota <= mid)).astype(jnp.int32), axis=1,
                          keepdims=True)
            pred = cnt >= t
            return jnp.where(pred, lo_, mid + 1), jnp.where(pred, mid, hi_)

        _, idx_cut = lax.fori_loop(0, n_idx_iters, ibody, (lo2, hi2))
        return idx_cut

    idx_cut = lax.cond(jnp.any(cnt_ge != kk), tie_path,
                       lambda: jnp.full((r, 1), hw - 1, jnp.int32))
    return vstar, idx_cut


def _count_select_i16(bits, kk, n_idx_iters, hw):
    r = bits.shape[0]

    ones128 = jnp.ones((128, 1), jnp.float32)

    def rowcount(m16):
        s = m16
        while s.shape[1] > 128:
            half = s.shape[1] // 2
            s = s[:, :half] + s[:, half:]
        return jnp.dot(s.astype(jnp.float32), ones128,
                       preferred_element_type=jnp.float32)

    hi = ((bits >> 15) - 32768).astype(jnp.int16)

    def vbody1(b, v):
        cand = v | (jnp.int32(1) << (15 - b))
        cand16 = (cand - 32768).astype(jnp.int16)
        cnt = rowcount((hi >= cand16).astype(jnp.int16))
        return jnp.where(cnt >= kk, cand, v)

    vhi = lax.fori_loop(0, 16, vbody1, jnp.zeros((r, 1), jnp.int32))
    vhi16 = (vhi - 32768).astype(jnp.int16)

    lo16 = (bits & 0x7FFF).astype(jnp.int16)
    y = jnp.where(hi > vhi16, jnp.int16(0x7FFF),
                  jnp.where(hi == vhi16, lo16, jnp.int16(-1)))

    def vbody2(b, v):
        cand = v | (jnp.int32(1) << (14 - b))
        cand16 = cand.astype(jnp.int16)
        cnt = rowcount((y >= cand16).astype(jnp.int16))
        return jnp.where(cnt >= kk, cand, v)

    vlo = lax.fori_loop(0, 15, vbody2, jnp.zeros((r, 1), jnp.int32))
    vstar = (vhi << 15) | vlo
    cnt_ge = rowcount((y >= vlo.astype(jnp.int16)).astype(jnp.int16))

    def tie_path():
        eq = bits == vstar
        cnt_gt = jnp.sum((bits > vstar).astype(jnp.int32), axis=1,
                         keepdims=True)
        t = kk - cnt_gt
        iota = lax.broadcasted_iota(jnp.int32, bits.shape, 1)
        lo2 = jnp.zeros((r, 1), jnp.int32)
        hi2 = jnp.full((r, 1), hw - 1, jnp.int32)

        def ibody(_, carry):
            lo_, hi_ = carry
            mid = lo_ + (hi_ - lo_) // 2
            cnt = jnp.sum((eq & (iota <= mid)).astype(jnp.int32), axis=1,
                          keepdims=True)
            pred = cnt >= t
            return jnp.where(pred, lo_, mid + 1), jnp.where(pred, mid, hi_)

        _, idx_cut = lax.fori_loop(0, n_idx_iters, ibody, (lo2, hi2))
        return idx_cut

    idx_cut = lax.cond(jnp.any(cnt_ge != kk), tie_path,
                       lambda: jnp.full((r, 1), hw - 1, jnp.int32))
    return vstar, idx_cut


def _phase_a(x_ref, vstar_ref, idxc_ref, chsum_ref, rowsum_ref, entx_ref,
             enty_ref, *, k, h, w, n_idx_iters):
    hw = h * w
    xb = x_ref[0]
    xa = jnp.abs(xb)
    bits = lax.bitcast_convert_type(xa, jnp.int32)
    vstar, idx_cut = _count_select_i16(bits, k, n_idx_iters, hw)

    iota = lax.broadcasted_iota(jnp.int32, bits.shape, 1)
    keep = (bits > vstar) | ((bits == vstar) & (iota <= idx_cut))

    chsum = jnp.sum(jnp.where(keep, xa, 0.0), axis=1, keepdims=True)
    rowsum = jnp.sum(xa, axis=1, keepdims=True)

    ii = lax.broadcasted_iota(jnp.int32, (hw, h), 0)
    jh = lax.broadcasted_iota(jnp.int32, (hw, h), 1)
    a_ind = ((ii // w) == jh).astype(jnp.float32)
    ii2 = lax.broadcasted_iota(jnp.int32, (hw, w), 0)
    jw = lax.broadcasted_iota(jnp.int32, (hw, w), 1)
    b_ind = ((ii2 % w) == jw).astype(jnp.float32)
    xcp = jnp.dot(xa, a_ind, preferred_element_type=jnp.float32)
    ycp = jnp.dot(xa, b_ind, preferred_element_type=jnp.float32)
    logs = jnp.log(rowsum)
    entx = logs - jnp.sum(xcp * jnp.log(xcp), axis=1, keepdims=True) / rowsum
    enty = logs - jnp.sum(ycp * jnp.log(ycp), axis=1, keepdims=True) / rowsum

    vstar_ref[0] = vstar
    idxc_ref[0] = idx_cut
    chsum_ref[0] = chsum
    rowsum_ref[0] = rowsum
    entx_ref[0] = entx
    enty_ref[0] = enty


def _phase_b(chsum_ref, rowsum_ref, entx_ref, enty_ref, chmask_ref, wcoef_ref,
             regp_ref, *, k2, c, hw, n_idx_iters):
    chsum = chsum_ref[...]
    rowsum = rowsum_ref[...]
    total = jnp.sum(chsum, axis=1, keepdims=True)
    chprob = chsum / total
    pbits = lax.bitcast_convert_type(chprob, jnp.int32)
    vstar, idx_cut = _count_select(pbits, k2, n_idx_iters, c)
    iota = lax.broadcasted_iota(jnp.int32, pbits.shape, 1)
    keep = (pbits > vstar) | ((pbits == vstar) & (iota <= idx_cut))
    chmask = keep.astype(jnp.float32)
    chmask_ref[...] = chmask
    wcoef_ref[...] = chmask / rowsum

    samplesum = jnp.sum(rowsum, axis=1, keepdims=True)
    wmag = rowsum / samplesum
    nrows = chsum.shape[0] * c
    mx = jnp.sum(entx_ref[...] * wmag) / nrows
    my = jnp.sum(enty_ref[...] * wmag) / nrows
    l1 = jnp.sum(rowsum) / (nrows * hw)
    regp = l1 * _LAMBDA_L1 - (mx + my) * _LAMBDA_LOCALITY
    regp_ref[...] = regp.reshape(1, 1)


def _phase_c(x_ref, vstar_ref, idxc_ref, chm_ref, wco_ref, regp_ref,
             sparse_ref, reg_ref, tacc_ref, cov_ref, *, nn, ncb, hw):
    n = pl.program_id(0)
    cb = pl.program_id(1)
    xb = x_ref[0]
    xa = jnp.abs(xb)
    bits = lax.bitcast_convert_type(xa, jnp.int32)
    vstar = vstar_ref[0]
    idx_cut = idxc_ref[0]
    chm = chm_ref[0]
    wco = wco_ref[0]
    iota = lax.broadcasted_iota(jnp.int32, bits.shape, 1)
    keep = (bits > vstar) | ((bits == vstar) & (iota <= idx_cut))
    sparse_ref[0] = xb * keep.astype(jnp.float32) * chm

    part = jnp.sum(xa * wco, axis=0, keepdims=True)

    @pl.when(cb == 0)
    def _():
        tacc_ref[0:1, :] = part

    @pl.when(cb != 0)
    def _():
        tacc_ref[0:1, :] = tacc_ref[0:1, :] + part

    @pl.when(cb == ncb - 1)
    def _():
        tt = tacc_ref[0:1, :]
        covn = jnp.sum((tt - 1.0) ** 2).reshape(1, 1)

        @pl.when(n == 0)
        def _():
            cov_ref[0:1, 0:1] = covn

        @pl.when(n != 0)
        def _():
            cov_ref[0:1, 0:1] = cov_ref[0:1, 0:1] + covn

        @pl.when(n == nn - 1)
        def _():
            reg_ref[...] = (regp_ref[...]
                            + cov_ref[0:1, 0:1] * (_LAMBDA_COV / (nn * hw)))


def kernel(x, tau):
    del tau
    n, c, h, w = x.shape
    hw = h * w
    rows = n * c
    k = max(int(_TOPK * hw), 1)
    k2 = max(int(_TOPK_CHANNEL * c), 1)
    r_blk = 1024 if rows % 1024 == 0 else rows
    nb = rows // r_blk
    it_hw = max((hw - 1).bit_length(), 1)
    it_c = max((c - 1).bit_length(), 1)

    xflat = x.reshape(nb, r_blk, hw)
    a_out = pl.pallas_call(
        functools.partial(_phase_a, k=k, h=h, w=w, n_idx_iters=it_hw),
        grid=(nb,),
        in_specs=[pl.BlockSpec((1, r_blk, hw), lambda i: (i, 0, 0))],
        out_specs=[pl.BlockSpec((1, r_blk, 1), lambda i: (i, 0, 0))] * 6,
        out_shape=[
            jax.ShapeDtypeStruct((nb, r_blk, 1), jnp.int32),
            jax.ShapeDtypeStruct((nb, r_blk, 1), jnp.int32),
            jax.ShapeDtypeStruct((nb, r_blk, 1), jnp.float32),
            jax.ShapeDtypeStruct((nb, r_blk, 1), jnp.float32),
            jax.ShapeDtypeStruct((nb, r_blk, 1), jnp.float32),
            jax.ShapeDtypeStruct((nb, r_blk, 1), jnp.float32),
        ],
    )(xflat)
    vstar, idx_cut, chsum, rowsum, entx, enty = a_out

    b_out = pl.pallas_call(
        functools.partial(_phase_b, k2=k2, c=c, hw=hw, n_idx_iters=it_c),
        out_shape=[
            jax.ShapeDtypeStruct((n, c), jnp.float32),
            jax.ShapeDtypeStruct((n, c), jnp.float32),
            jax.ShapeDtypeStruct((1, 1), jnp.float32),
        ],
    )(chsum.reshape(n, c), rowsum.reshape(n, c), entx.reshape(n, c),
      enty.reshape(n, c))
    chmask, wcoef, regpart = b_out

    cblk = 256 if c % 256 == 0 else c
    ncb = c // cblk
    nrb = rows // cblk
    x3 = x.reshape(n, c, hw)
    sparse, reg = pl.pallas_call(
        functools.partial(_phase_c, nn=n, ncb=ncb, hw=hw),
        grid=(n, ncb),
        in_specs=[
            pl.BlockSpec((1, cblk, hw), lambda i, j: (i, j, 0)),
            pl.BlockSpec((1, cblk, 1), lambda i, j, _ncb=ncb: (i * _ncb + j, 0, 0)),
            pl.BlockSpec((1, cblk, 1), lambda i, j, _ncb=ncb: (i * _ncb + j, 0, 0)),
            pl.BlockSpec((1, cblk, 1), lambda i, j, _ncb=ncb: (i * _ncb + j, 0, 0)),
            pl.BlockSpec((1, cblk, 1), lambda i, j, _ncb=ncb: (i * _ncb + j, 0, 0)),
            pl.BlockSpec((1, 1), lambda i, j: (0, 0)),
        ],
        out_specs=[
            pl.BlockSpec((1, cblk, hw), lambda i, j: (i, j, 0)),
            pl.BlockSpec((1, 1), lambda i, j: (0, 0)),
        ],
        out_shape=[
            jax.ShapeDtypeStruct((n, c, hw), jnp.float32),
            jax.ShapeDtypeStruct((1, 1), jnp.float32),
        ],
        scratch_shapes=[
            pltpu.VMEM((8, hw), jnp.float32),
            pltpu.VMEM((8, 128), jnp.float32),
        ],
    )(x3, vstar.reshape(nrb, cblk, 1), idx_cut.reshape(nrb, cblk, 1),
      chmask.reshape(nrb, cblk, 1), wcoef.reshape(nrb, cblk, 1), regpart)

    return sparse.reshape(n, c, h, w), reg[0, 0]
